# baseline (speedup 1.0000x reference)
import functools

import jax
import jax.numpy as jnp
import numpy as np
from jax import lax
from jax.experimental import pallas as pl
from jax.experimental.pallas import tpu as pltpu
from jax.experimental.pallas import tpu_sc as plsc

BLOCK = 128
NB = 16
H = 16
S = 2048
D = 128
SCALE = 1.0 / np.sqrt(D)
LOG2E = float(np.log2(np.e))
MIN_RETAIN = 1
MAX_RETAIN = 11
ENERGY = 0.95
QTILE = 2048
NQT = S // QTILE
QB_PER_TILE = QTILE // BLOCK
KCHUNK = 512
NKC = S // KCHUNK
KB_PER_CHUNK = KCHUNK // BLOCK

ROWS = H * NB
SC_WORKERS = 32
RPW = ROWS // SC_WORKERS


def _prep_kernel(q_ref, k_ref, v_ref, sp_ref, cb_ref, qb_ref, kb_ref, vb_ref):
    q = q_ref[0]
    k = k_ref[0]
    qb_ref[0] = (q * (SCALE * LOG2E)).astype(jnp.bfloat16)
    kb_ref[0] = k.astype(jnp.bfloat16)
    vb_ref[0] = v_ref[0].astype(jnp.bfloat16)

    qp = q.reshape(NB, BLOCK, D).mean(axis=1)
    kp = k.reshape(NB, BLOCK, D).mean(axis=1)
    sraw = jax.lax.dot_general(qp, kp, (((1,), (1,)), ((), ())),
                               preferred_element_type=jnp.float32) * SCALE
    sp_ref[0] = sraw - sraw.max(axis=-1, keepdims=True)

    qsq = jnp.sum(q * q, axis=1).reshape(NB, BLOCK)
    qbmax = qsq.max(axis=1, keepdims=True)
    kmax = jnp.max(jnp.sum(k * k, axis=1))
    cb_ref[0] = (SCALE * jnp.sqrt(qbmax * kmax)).reshape(1, NB)


def _sc_mask_body(sp_ref, cb_ref, out_ref, srow, cbv, outv):
    w = lax.axis_index("s") * 2 + lax.axis_index("c")
    base = w * RPW
    pltpu.sync_copy(sp_ref.at[pl.ds(base, RPW)], srow)
    pltpu.sync_copy(cb_ref.at[pl.ds(base, RPW)], cbv.at[pl.ds(0, RPW)])
    cbvec = cbv[...]
    for r in range(RPW):
        row = srow[r]
        e = jnp.exp(row)
        idx = lax.iota(jnp.int32, 16)
        sk, sv = plsc.sort_key_val(e, idx, descending=True)
        cum = plsc.cumsum(sk)
        tot = cum[15]
        cnt_v = plsc.all_reduce_population_count(cum < ENERGY * tot)
        kk = jnp.clip(cnt_v[0], MIN_RETAIN, MAX_RETAIN)
        cbs = cbvec[r]
        vals = jnp.where(idx < kk, cbs, cbs + 1e30) * LOG2E
        plsc.store_scatter(outv, [jnp.full((16,), r, jnp.int32), sv], vals)
    pltpu.sync_copy(outv, out_ref.at[pl.ds(base, RPW)])


_sc_mask = pl.kernel(
    _sc_mask_body,
    out_type=jax.ShapeDtypeStruct((ROWS, NB), jnp.float32),
    mesh=plsc.VectorSubcoreMesh(core_axis_name="c", subcore_axis_name="s"),
    compiler_params=pltpu.CompilerParams(needs_layout_passes=False),
    scratch_types=[
        pltpu.VMEM((RPW, NB), jnp.float32),
        pltpu.VMEM((16,), jnp.float32),
        pltpu.VMEM((RPW, NB), jnp.float32),
    ],
)


def _attn_kernel(q_ref, k_ref, v_ref, cbm_ref, o_ref):
    q = q_ref[0]
    cbm = cbm_ref[0, 0]
    l = None
    acc = None
    for c in range(NKC):
        kc = k_ref[0, c * KCHUNK:(c + 1) * KCHUNK, :]
        s = jax.lax.dot_general(q, kc, (((1,), (1,)), ((), ())),
                                preferred_element_type=jnp.float32)
        sub = jnp.repeat(cbm[:, c * KB_PER_CHUNK:(c + 1) * KB_PER_CHUNK],
                         BLOCK, axis=1)
        pf = jnp.exp2(s.reshape(QB_PER_TILE, BLOCK, KCHUNK)
                      - sub[:, None, :]).reshape(QTILE, KCHUNK)
        ls = pf.sum(axis=1, keepdims=True)
        vc = v_ref[0, c * KCHUNK:(c + 1) * KCHUNK, :]
        pv = jax.lax.dot_general(pf.astype(jnp.bfloat16), vc,
                                 (((1,), (0,)), ((), ())),
                                 preferred_element_type=jnp.float32)
        if c == 0:
            l = ls
            acc = pv
        else:
            l = l + ls
            acc = acc + pv
    o_ref[0] = acc / l


@functools.partial(jax.jit, static_argnames=("interpret",))
def _run(q3, k3, v3, interpret=False):
    sp, cb, qb, kb, vb = pl.pallas_call(
        _prep_kernel,
        grid=(H,),
        in_specs=[
            pl.BlockSpec((1, S, D), lambda h: (h, 0, 0)),
            pl.BlockSpec((1, S, D), lambda h: (h, 0, 0)),
            pl.BlockSpec((1, S, D), lambda h: (h, 0, 0)),
        ],
        out_specs=[
            pl.BlockSpec((1, NB, NB), lambda h: (h, 0, 0)),
            pl.BlockSpec((1, 1, NB), lambda h: (h, 0, 0)),
            pl.BlockSpec((1, S, D), lambda h: (h, 0, 0)),
            pl.BlockSpec((1, S, D), lambda h: (h, 0, 0)),
            pl.BlockSpec((1, S, D), lambda h: (h, 0, 0)),
        ],
        out_shape=[
            jax.ShapeDtypeStruct((H, NB, NB), jnp.float32),
            jax.ShapeDtypeStruct((H, 1, NB), jnp.float32),
            jax.ShapeDtypeStruct((H, S, D), jnp.bfloat16),
            jax.ShapeDtypeStruct((H, S, D), jnp.bfloat16),
            jax.ShapeDtypeStruct((H, S, D), jnp.bfloat16),
        ],
        compiler_params=pltpu.CompilerParams(
            dimension_semantics=("parallel",)),
        interpret=interpret,
    )(q3, k3, v3)

    cbm = _sc_mask(sp.reshape(ROWS, NB), cb.reshape(ROWS))

    cbm4 = cbm.reshape(H, NQT, QB_PER_TILE, NB)

    o3 = pl.pallas_call(
        _attn_kernel,
        grid=(H, NQT),
        in_specs=[
            pl.BlockSpec((1, QTILE, D), lambda h, i: (h, i, 0)),
            pl.BlockSpec((1, S, D), lambda h, i: (h, 0, 0)),
            pl.BlockSpec((1, S, D), lambda h, i: (h, 0, 0)),
            pl.BlockSpec((1, 1, QB_PER_TILE, NB), lambda h, i: (h, i, 0, 0)),
        ],
        out_specs=pl.BlockSpec((1, QTILE, D), lambda h, i: (h, i, 0)),
        out_shape=jax.ShapeDtypeStruct((H, S, D), jnp.float32),
        compiler_params=pltpu.CompilerParams(
            dimension_semantics=("parallel", "parallel")),
        interpret=interpret,
    )(qb, kb, vb, cbm4)
    return o3


def kernel(q, k, v):
    q3 = q[0]
    k3 = k[0]
    v3 = v[0]
    return _run(q3, k3, v3)[None]


# speedup vs baseline: 1.0668x; 1.0668x over previous
import functools

import jax
import jax.numpy as jnp
import numpy as np
from jax import lax
from jax.experimental import pallas as pl
from jax.experimental.pallas import tpu as pltpu
from jax.experimental.pallas import tpu_sc as plsc

BLOCK = 128
NB = 16
H = 16
S = 2048
D = 128
SCALE = 1.0 / np.sqrt(D)
LOG2E = float(np.log2(np.e))
MIN_RETAIN = 1
MAX_RETAIN = 11
ENERGY = 0.95
QTILE = 2048
NQT = S // QTILE
QB_PER_TILE = QTILE // BLOCK
KCHUNK = 256
NKC = S // KCHUNK
KB_PER_CHUNK = KCHUNK // BLOCK

ROWS = H * NB
SC_WORKERS = 32
RPW = ROWS // SC_WORKERS


def _prep_kernel(q_ref, k_ref, v_ref, sp_ref, cb_ref, qb_ref, kb_ref, vb_ref):
    q = q_ref[0]
    k = k_ref[0]
    qb_ref[0] = (q * (SCALE * LOG2E)).astype(jnp.bfloat16)
    kb_ref[0] = k.astype(jnp.bfloat16)
    vb_ref[0] = v_ref[0].astype(jnp.bfloat16)

    qp = q.reshape(NB, BLOCK, D).mean(axis=1)
    kp = k.reshape(NB, BLOCK, D).mean(axis=1)
    sraw = jax.lax.dot_general(qp, kp, (((1,), (1,)), ((), ())),
                               preferred_element_type=jnp.float32) * SCALE
    sp_ref[0] = sraw - sraw.max(axis=-1, keepdims=True)

    qsq = jnp.sum(q * q, axis=1).reshape(NB, BLOCK)
    qbmax = qsq.max(axis=1, keepdims=True)
    kmax = jnp.max(jnp.sum(k * k, axis=1))
    cb_ref[0] = (SCALE * jnp.sqrt(qbmax * kmax)).reshape(1, NB)


def _sc_mask_body(sp_ref, cb_ref, out_ref, srow, cbv, outv):
    w = lax.axis_index("s") * 2 + lax.axis_index("c")
    base = w * RPW
    pltpu.sync_copy(sp_ref.at[pl.ds(base, RPW)], srow)
    pltpu.sync_copy(cb_ref.at[pl.ds(base, RPW)], cbv.at[pl.ds(0, RPW)])
    cbvec = cbv[...]
    for r in range(RPW):
        row = srow[r]
        e = jnp.exp(row)
        idx = lax.iota(jnp.int32, 16)
        sk, sv = plsc.sort_key_val(e, idx, descending=True)
        cum = plsc.cumsum(sk)
        tot = cum[15]
        cnt_v = plsc.all_reduce_population_count(cum < ENERGY * tot)
        kk = jnp.clip(cnt_v[0], MIN_RETAIN, MAX_RETAIN)
        cbs = cbvec[r]
        vals = jnp.where(idx < kk, cbs, cbs + 1e30) * LOG2E
        plsc.store_scatter(outv, [jnp.full((16,), r, jnp.int32), sv], vals)
    pltpu.sync_copy(outv, out_ref.at[pl.ds(base, RPW)])


_sc_mask = pl.kernel(
    _sc_mask_body,
    out_type=jax.ShapeDtypeStruct((ROWS, NB), jnp.float32),
    mesh=plsc.VectorSubcoreMesh(core_axis_name="c", subcore_axis_name="s"),
    compiler_params=pltpu.CompilerParams(needs_layout_passes=False),
    scratch_types=[
        pltpu.VMEM((RPW, NB), jnp.float32),
        pltpu.VMEM((16,), jnp.float32),
        pltpu.VMEM((RPW, NB), jnp.float32),
    ],
)


def _attn_kernel(q_ref, k_ref, v_ref, cbm_ref, o_ref):
    q = q_ref[0]
    cbm = cbm_ref[0, 0]
    l = None
    acc = None
    for c in range(NKC):
        kc = k_ref[0, c * KCHUNK:(c + 1) * KCHUNK, :]
        s = jax.lax.dot_general(q, kc, (((1,), (1,)), ((), ())),
                                preferred_element_type=jnp.float32)
        sub = jnp.repeat(cbm[:, c * KB_PER_CHUNK:(c + 1) * KB_PER_CHUNK],
                         BLOCK, axis=1)
        pf = jnp.exp2(s.reshape(QB_PER_TILE, BLOCK, KCHUNK)
                      - sub[:, None, :]).reshape(QTILE, KCHUNK)
        ls = pf.sum(axis=1, keepdims=True)
        vc = v_ref[0, c * KCHUNK:(c + 1) * KCHUNK, :]
        pv = jax.lax.dot_general(pf.astype(jnp.bfloat16), vc,
                                 (((1,), (0,)), ((), ())),
                                 preferred_element_type=jnp.float32)
        if c == 0:
            l = ls
            acc = pv
        else:
            l = l + ls
            acc = acc + pv
    o_ref[0] = acc / l


@functools.partial(jax.jit, static_argnames=("interpret",))
def _run(q3, k3, v3, interpret=False):
    sp, cb, qb, kb, vb = pl.pallas_call(
        _prep_kernel,
        grid=(H,),
        in_specs=[
            pl.BlockSpec((1, S, D), lambda h: (h, 0, 0)),
            pl.BlockSpec((1, S, D), lambda h: (h, 0, 0)),
            pl.BlockSpec((1, S, D), lambda h: (h, 0, 0)),
        ],
        out_specs=[
            pl.BlockSpec((1, NB, NB), lambda h: (h, 0, 0)),
            pl.BlockSpec((1, 1, NB), lambda h: (h, 0, 0)),
            pl.BlockSpec((1, S, D), lambda h: (h, 0, 0)),
            pl.BlockSpec((1, S, D), lambda h: (h, 0, 0)),
            pl.BlockSpec((1, S, D), lambda h: (h, 0, 0)),
        ],
        out_shape=[
            jax.ShapeDtypeStruct((H, NB, NB), jnp.float32),
            jax.ShapeDtypeStruct((H, 1, NB), jnp.float32),
            jax.ShapeDtypeStruct((H, S, D), jnp.bfloat16),
            jax.ShapeDtypeStruct((H, S, D), jnp.bfloat16),
            jax.ShapeDtypeStruct((H, S, D), jnp.bfloat16),
        ],
        compiler_params=pltpu.CompilerParams(
            dimension_semantics=("parallel",)),
        interpret=interpret,
    )(q3, k3, v3)

    cbm = _sc_mask(sp.reshape(ROWS, NB), cb.reshape(ROWS))

    cbm4 = cbm.reshape(H, NQT, QB_PER_TILE, NB)

    o3 = pl.pallas_call(
        _attn_kernel,
        grid=(H, NQT),
        in_specs=[
            pl.BlockSpec((1, QTILE, D), lambda h, i: (h, i, 0)),
            pl.BlockSpec((1, S, D), lambda h, i: (h, 0, 0)),
            pl.BlockSpec((1, S, D), lambda h, i: (h, 0, 0)),
            pl.BlockSpec((1, 1, QB_PER_TILE, NB), lambda h, i: (h, i, 0, 0)),
        ],
        out_specs=pl.BlockSpec((1, QTILE, D), lambda h, i: (h, i, 0)),
        out_shape=jax.ShapeDtypeStruct((H, S, D), jnp.float32),
        compiler_params=pltpu.CompilerParams(
            dimension_semantics=("parallel", "parallel")),
        interpret=interpret,
    )(qb, kb, vb, cbm4)
    return o3


def kernel(q, k, v):
    q3 = q[0]
    k3 = k[0]
    v3 = v[0]
    return _run(q3, k3, v3)[None]
